# Initial kernel scaffold; baseline (speedup 1.0000x reference)
import jax
import jax.numpy as jnp
from jax.experimental import pallas as pl


def kernel(x, U, Wr, spec_bias, gen_bias, Wg, Wu, Wd, Sg, Su, Sd, norm_w):
    raise NotImplementedError("write your pallas kernel here")


# speedup vs baseline: 1.1435x; 1.1435x over previous
import functools

import jax
import jax.numpy as jnp
from jax.experimental import pallas as pl
from jax.experimental.pallas import tpu as pltpu

_T = 2048
_D = 1024
_F = 1024
_E = 8
_NE = 9
_LOAD_COEF = 0.01
_EPS = 1e-6
_TB = 512


def _router_kernel(flat_ref, u_ref, wr_ref, sb_ref, gb_ref, w_ref, bal_ref):
    flat = flat_ref[...]
    logits = jax.lax.dot_general(
        flat, wr_ref[...], (((1,), (1,)), ((), ())),
        preferred_element_type=jnp.float32)
    u = jnp.clip(u_ref[...], 0.0, 1.0)
    logits = logits + u * sb_ref[...] + (1.0 - u) * gb_ref[...]
    m = jnp.max(logits, axis=-1, keepdims=True)
    ex = jnp.exp(logits - m)
    probs = ex / jnp.sum(ex, axis=-1, keepdims=True)

    lane = jax.lax.broadcasted_iota(jnp.int32, probs.shape, 1)
    m1 = jnp.max(probs, axis=-1, keepdims=True)
    i1 = jnp.min(jnp.where(probs == m1, lane, _E), axis=-1, keepdims=True)
    oh1 = (lane == i1)
    p_rest = jnp.where(oh1, -jnp.inf, probs)
    m2 = jnp.max(p_rest, axis=-1, keepdims=True)
    i2 = jnp.min(jnp.where(p_rest == m2, lane, _E), axis=-1, keepdims=True)
    oh2 = (lane == i2)
    w = jnp.where(oh1, m1, 0.0) + jnp.where(oh2, m2, 0.0)

    w_ref[:, :_E] = w
    w_ref[:, _E:] = jnp.ones((_T, 1), jnp.float32)
    colmean = jnp.mean(probs, axis=0, keepdims=True)
    bal_ref[...] = _E * _LOAD_COEF * jnp.sum(
        colmean * colmean, keepdims=True).reshape(1, 1)


def _expert_kernel(w_ref, x_ref, wg_ref, wu_ref, wd_ref, nw_ref, out_ref,
                   acc_ref):
    e = pl.program_id(0)
    t = pl.program_id(1)
    x = x_ref[...]
    g = jax.lax.dot_general(x, wg_ref[0], (((1,), (1,)), ((), ())),
                            preferred_element_type=jnp.float32)
    u = jax.lax.dot_general(x, wu_ref[0], (((1,), (1,)), ((), ())),
                            preferred_element_type=jnp.float32)
    h = (g * jax.lax.logistic(g)) * u
    eo = jax.lax.dot_general(h, wd_ref[0], (((1,), (1,)), ((), ())),
                             preferred_element_type=jnp.float32)
    w_full = w_ref[...]
    lane = jax.lax.broadcasted_iota(jnp.int32, w_full.shape, 1)
    w_col = jnp.sum(jnp.where(lane == e, w_full, 0.0), axis=1, keepdims=True)
    contrib = w_col * eo
    rows = pl.ds(t * _TB, _TB)

    @pl.when(e == 0)
    def _():
        acc_ref[rows, :] = contrib

    @pl.when((e > 0) & (e < _NE - 1))
    def _():
        acc_ref[rows, :] += contrib

    @pl.when(e == _NE - 1)
    def _():
        y = acc_ref[rows, :] + contrib
        rms = jax.lax.rsqrt(jnp.mean(y * y, axis=-1, keepdims=True) + _EPS)
        out_ref[...] = y * rms * nw_ref[...]


@jax.jit
def kernel(x, U, Wr, spec_bias, gen_bias, Wg, Wu, Wd, Sg, Su, Sd, norm_w):
    flat = x.reshape(_T, _D)
    u_col = U.reshape(_T, 1)

    w_all, bal = pl.pallas_call(
        _router_kernel,
        out_shape=(
            jax.ShapeDtypeStruct((_T, _NE), jnp.float32),
            jax.ShapeDtypeStruct((1, 1), jnp.float32),
        ),
    )(flat, u_col, Wr, spec_bias.reshape(1, _E), gen_bias.reshape(1, _E))

    wg_all = jnp.concatenate([Wg, Sg], axis=0)
    wu_all = jnp.concatenate([Wu, Su], axis=0)
    wd_all = jnp.concatenate([Wd, Sd], axis=0)

    nt = _T // _TB
    y = pl.pallas_call(
        _expert_kernel,
        grid=(_NE, nt),
        in_specs=[
            pl.BlockSpec((_TB, _NE), lambda e, t: (t, 0)),
            pl.BlockSpec((_TB, _D), lambda e, t: (t, 0)),
            pl.BlockSpec((1, _F, _D), lambda e, t: (e, 0, 0)),
            pl.BlockSpec((1, _F, _D), lambda e, t: (e, 0, 0)),
            pl.BlockSpec((1, _D, _F), lambda e, t: (e, 0, 0)),
            pl.BlockSpec((1, _D), lambda e, t: (0, 0)),
        ],
        out_specs=pl.BlockSpec(
            (_TB, _D), lambda e, t: (jnp.where(e == _NE - 1, t, 0), 0)),
        out_shape=jax.ShapeDtypeStruct((_T, _D), jnp.float32),
        scratch_shapes=[pltpu.VMEM((_T, _D), jnp.float32)],
    )(w_all, flat, wg_all, wu_all, wd_all, norm_w.reshape(1, _D))

    return (y.reshape(x.shape), bal.reshape(()))
